# baseline (speedup 1.0000x reference)
import functools

import jax
import jax.numpy as jnp
from jax import lax
from jax.experimental import pallas as pl
from jax.experimental.pallas import tpu as pltpu
from jax.experimental.pallas import tpu_sc as plsc

N_NODES = 10000
N_EDGES = 320000
D_IN = 128
D_OUT = 128
D_HALF = D_OUT // 2

NC = 2
NS = 16
CHUNK = 128
NCHUNK = 162
EPW = NCHUNK * CHUNK
E_PAD = NS * EPW
NBUF = 3
ROWS_PT = 624
ROWS_LAST = N_NODES - (NS - 1) * ROWS_PT
LANES = 16
VPR = D_HALF // LANES


def _mm_body(x_ref, w_ref, o_ref):
    o_ref[...] = jnp.dot(x_ref[...], w_ref[0],
                         preferred_element_type=jnp.float32)


def _matmul(x, W):
    grid_i = 10
    rows = N_NODES // grid_i
    Wr = jnp.stack([W[:, :D_HALF], W[:, D_HALF:]])
    return pl.pallas_call(
        _mm_body,
        grid=(grid_i, NC),
        in_specs=[
            pl.BlockSpec((rows, D_IN), lambda i, j: (i, 0)),
            pl.BlockSpec((1, D_IN, D_HALF), lambda i, j: (j, 0, 0)),
        ],
        out_specs=pl.BlockSpec((rows, D_HALF), lambda i, j: (j * grid_i + i, 0)),
        out_shape=jax.ShapeDtypeStruct((NC * N_NODES, D_HALF), jnp.float32),
    )(x, Wr)


_mesh = plsc.VectorSubcoreMesh(core_axis_name="c", subcore_axis_name="s")


@functools.partial(
    pl.kernel,
    out_type=jax.ShapeDtypeStruct((NC, N_NODES, D_HALF), jnp.float32),
    mesh=_mesh,
    scratch_types=[
        pltpu.VMEM((NCHUNK, CHUNK), jnp.int32),
        pltpu.VMEM((NCHUNK, CHUNK), jnp.int32),
        pltpu.VMEM((NCHUNK, CHUNK), jnp.float32),
        pltpu.VMEM((CHUNK, D_HALF), jnp.float32),
        pltpu.VMEM((CHUNK, D_HALF), jnp.float32),
        pltpu.VMEM((CHUNK, D_HALF), jnp.float32),
        pltpu.VMEM_SHARED((N_NODES, D_HALF), jnp.float32),
        pltpu.SemaphoreType.DMA,
        pltpu.SemaphoreType.DMA,
        pltpu.SemaphoreType.DMA,
        pltpu.SemaphoreType.DMA,
        pltpu.SemaphoreType.DMA,
        pltpu.SemaphoreType.DMA,
    ],
    compiler_params=pltpu.CompilerParams(use_tc_tiling_on_sc=False),
)
def _sc_edges(src_hbm, dst_hbm, w_hbm, h_hbm, out_hbm,
              src_v, dst_v, w_v, rows0, rows1, rows2, acc_sh,
              gat0, gat1, gat2, scat0, scat1, scat2):
    cid = lax.axis_index("c")
    sid = lax.axis_index("s")
    rows = (rows0, rows1, rows2)
    gat = (gat0, gat1, gat2)
    scat = (scat0, scat1, scat2)

    pltpu.sync_copy(src_hbm.at[cid, sid], src_v)
    pltpu.sync_copy(dst_hbm.at[sid], dst_v)
    pltpu.sync_copy(w_hbm.at[sid], w_v)

    zvec = jnp.zeros((LANES,), jnp.float32)

    def _zero_row(r, _):
        for j in range(VPR):
            rows0[r, pl.ds(j * LANES, LANES)] = zvec
        return 0

    lax.fori_loop(0, CHUNK, _zero_row, 0)
    row_base = pl.multiple_of(sid * ROWS_PT, 8)
    nfull = ROWS_PT // CHUNK
    rem = ROWS_PT - nfull * CHUNK
    rem_last = ROWS_LAST - nfull * CHUNK
    for z in range(nfull):
        pltpu.sync_copy(rows0,
                        acc_sh.at[pl.ds(row_base + z * CHUNK, CHUNK)])

    @pl.when(sid < NS - 1)
    def _zero_tail():
        pltpu.sync_copy(rows0.at[pl.ds(0, rem)],
                        acc_sh.at[pl.ds(row_base + nfull * CHUNK, rem)])

    @pl.when(sid == NS - 1)
    def _zero_tail_last():
        pltpu.sync_copy(rows0.at[pl.ds(0, rem_last)],
                        acc_sh.at[pl.ds((NS - 1) * ROWS_PT + nfull * CHUNK,
                                        rem_last)])

    def _gather(i, b):
        pltpu.async_copy(h_hbm.at[src_v.at[i]], rows[b], gat[b])

    def _scatter(i, b):
        pltpu.async_copy(rows[b], acc_sh.at[dst_v.at[i]], scat[b], add=True)

    def _scale(i, b):
        def _group(g, _):
            wv = w_v[i, pl.ds(g * LANES, LANES)]
            for t in range(LANES):
                e = g * LANES + t
                w = wv[t]
                for j in range(VPR):
                    sl = pl.ds(j * LANES, LANES)
                    rows[b][e, sl] = rows[b][e, sl] * w
            return 0

        lax.fori_loop(0, CHUNK // LANES, _group, 0)

    for b in range(NBUF - 1):
        _gather(b, b)

    plsc.subcore_barrier()

    def _slot(i, b):
        pltpu.make_async_copy(h_hbm.at[src_v.at[i]], rows[b], gat[b]).wait()
        _scale(i, b)
        _scatter(i, b)

        @pl.when(i + NBUF - 1 < NCHUNK)
        def _refill():
            nb = (b + NBUF - 1) % NBUF

            @pl.when(i > 0)
            def _drain_prev():
                pltpu.make_async_copy(rows[nb], acc_sh.at[dst_v.at[i]],
                                      scat[nb]).wait()

            _gather(i + NBUF - 1, nb)

    def _ring(t, _):
        for k in range(NBUF):
            _slot(t * NBUF + k, k)
        return 0

    lax.fori_loop(0, NCHUNK // NBUF, _ring, 0)

    for b in range(NBUF):
        pltpu.make_async_copy(rows[b], acc_sh.at[dst_v.at[0]],
                              scat[b]).wait()
    plsc.subcore_barrier()

    @pl.when(sid < NS - 1)
    def _wb_main():
        pltpu.sync_copy(acc_sh.at[pl.ds(row_base, ROWS_PT)],
                        out_hbm.at[cid, pl.ds(row_base, ROWS_PT)])

    @pl.when(sid == NS - 1)
    def _wb_last():
        last = (NS - 1) * ROWS_PT
        pltpu.sync_copy(acc_sh.at[pl.ds(last, ROWS_LAST)],
                        out_hbm.at[cid, pl.ds(last, ROWS_LAST)])


def kernel(x, edge_index, edge_weight, W):
    edge_index = edge_index.astype(jnp.int32)
    pad = E_PAD - N_EDGES
    src = jnp.concatenate(
        [edge_index[0], jnp.zeros((pad,), jnp.int32)]).reshape(
            NS, NCHUNK, CHUNK)
    src = jnp.stack([src, src + N_NODES])
    dst = jnp.concatenate(
        [edge_index[1], jnp.zeros((pad,), jnp.int32)]).reshape(
            NS, NCHUNK, CHUNK)
    ew = jnp.concatenate(
        [edge_weight, jnp.zeros((pad,), jnp.float32)]).reshape(
            NS, NCHUNK, CHUNK)
    h = _matmul(x, W)
    halves = _sc_edges(src, dst, ew, h)
    return jnp.concatenate([halves[0], halves[1]], axis=1)


# speedup vs baseline: 1.0804x; 1.0804x over previous
import functools

import jax
import jax.numpy as jnp
from jax import lax
from jax.experimental import pallas as pl
from jax.experimental.pallas import tpu as pltpu
from jax.experimental.pallas import tpu_sc as plsc

N_NODES = 10000
N_EDGES = 320000
D_IN = 128
D_OUT = 128

NC = 2
NS = 16
NW = NC * NS
CHUNK = 128
G = 10
NG = 8
NCHUNK = G * NG
EPW = NCHUNK * CHUNK
E_PAD = NW * EPW
NBUF = 2
ROWS_PT = 624
ROWS_LAST = N_NODES - (NS - 1) * ROWS_PT
LANES = 16
VPR = D_OUT // LANES


def _mm_body(x_ref, w_ref, o_ref):
    o_ref[...] = jnp.dot(x_ref[...], w_ref[...],
                         preferred_element_type=jnp.float32)


def _matmul(x, W):
    grid = 10
    rows = N_NODES // grid
    return pl.pallas_call(
        _mm_body,
        grid=(grid,),
        in_specs=[
            pl.BlockSpec((rows, D_IN), lambda i: (i, 0)),
            pl.BlockSpec((D_IN, D_OUT), lambda i: (0, 0)),
        ],
        out_specs=pl.BlockSpec((rows, D_OUT), lambda i: (i, 0)),
        out_shape=jax.ShapeDtypeStruct((N_NODES, D_OUT), jnp.float32),
    )(x, W)


_mesh = plsc.VectorSubcoreMesh(core_axis_name="c", subcore_axis_name="s")


@functools.partial(
    pl.kernel,
    out_type=jax.ShapeDtypeStruct((NC, N_NODES, D_OUT), jnp.float32),
    mesh=_mesh,
    scratch_types=[
        pltpu.VMEM((G, CHUNK), jnp.int32),
        pltpu.VMEM((G, CHUNK), jnp.int32),
        pltpu.VMEM((G, CHUNK), jnp.float32),
        pltpu.VMEM((G, CHUNK), jnp.int32),
        pltpu.VMEM((G, CHUNK), jnp.int32),
        pltpu.VMEM((G, CHUNK), jnp.float32),
        pltpu.VMEM((CHUNK, D_OUT), jnp.float32),
        pltpu.VMEM((CHUNK, D_OUT), jnp.float32),
        pltpu.VMEM_SHARED((N_NODES, D_OUT), jnp.float32),
        pltpu.SemaphoreType.DMA,
        pltpu.SemaphoreType.DMA,
        pltpu.SemaphoreType.DMA,
        pltpu.SemaphoreType.DMA,
        pltpu.SemaphoreType.DMA,
        pltpu.SemaphoreType.DMA,
    ],
)
def _sc_edges(src_hbm, dst_hbm, w_hbm, h_hbm, out_hbm,
              srcA, dstA, wA, srcB, dstB, wB, rows0, rows1, acc_sh,
              gat0, gat1, scat0, scat1, isemA, isemB):
    cid = lax.axis_index("c")
    sid = lax.axis_index("s")
    wid = sid * NC + cid
    rows = (rows0, rows1)
    gat = (gat0, gat1)
    scat = (scat0, scat1)
    srcg = (srcA, srcB)
    dstg = (dstA, dstB)
    wg = (wA, wB)
    isem = (isemA, isemB)

    def _load_group(g, a, sync):
        bufs = (srcg[a], dstg[a], wg[a])
        hbms = (src_hbm, dst_hbm, w_hbm)
        for hb, vb in zip(hbms, bufs):
            if sync:
                pltpu.sync_copy(hb.at[wid, g], vb)
            else:
                pltpu.async_copy(hb.at[wid, g], vb, isem[a])

    def _wait_group(g, a):
        bufs = (srcg[a], dstg[a], wg[a])
        hbms = (src_hbm, dst_hbm, w_hbm)
        for hb, vb in zip(hbms, bufs):
            pltpu.make_async_copy(hb.at[wid, g], vb, isem[a]).wait()

    zvec = jnp.zeros((LANES,), jnp.float32)

    def _zero_row(r, _):
        for j in range(VPR):
            rows0[r, pl.ds(j * LANES, LANES)] = zvec
        return 0

    lax.fori_loop(0, CHUNK, _zero_row, 0)
    row_base = pl.multiple_of(sid * ROWS_PT, 8)
    nfull = ROWS_PT // CHUNK
    rem = ROWS_PT - nfull * CHUNK
    rem_last = ROWS_LAST - nfull * CHUNK
    for z in range(nfull):
        pltpu.sync_copy(rows0,
                        acc_sh.at[pl.ds(row_base + z * CHUNK, CHUNK)])

    @pl.when(sid < NS - 1)
    def _zero_tail():
        pltpu.sync_copy(rows0.at[pl.ds(0, rem)],
                        acc_sh.at[pl.ds(row_base + nfull * CHUNK, rem)])

    @pl.when(sid == NS - 1)
    def _zero_tail_last():
        pltpu.sync_copy(rows0.at[pl.ds(0, rem_last)],
                        acc_sh.at[pl.ds((NS - 1) * ROWS_PT + nfull * CHUNK,
                                        rem_last)])

    def _gather(sref, r, b):
        pltpu.async_copy(h_hbm.at[sref.at[r]], rows[b], gat[b])

    def _wait_gather(sref, r, b):
        pltpu.make_async_copy(h_hbm.at[sref.at[r]], rows[b], gat[b]).wait()

    def _scatter(dref, r, b):
        pltpu.async_copy(rows[b], acc_sh.at[dref.at[r]], scat[b], add=True)

    def _wait_scatter(dref, r, b):
        pltpu.make_async_copy(rows[b], acc_sh.at[dref.at[r]], scat[b]).wait()

    def _scale(wref, r, b):
        def _group(g, _):
            wv = wref[r, pl.ds(g * LANES, LANES)]
            for t in range(LANES):
                e = g * LANES + t
                w = wv[t]
                for j in range(VPR):
                    sl = pl.ds(j * LANES, LANES)
                    rows[b][e, sl] = rows[b][e, sl] * w
            return 0

        lax.fori_loop(0, CHUNK // LANES, _group, 0)

    _load_group(0, 0, True)
    _gather(srcA, 0, 0)

    plsc.subcore_barrier()

    def _dgroup(d, _):
        for m in range(2 * G):
            i = d * (2 * G) + m
            b = m % NBUF
            nb = (b + 1) % NBUF
            a_cur = 0 if m < G else 1
            r_cur = m if m < G else m - G
            if m + 1 < G:
                a_nxt, r_nxt = 0, m + 1
            elif m + 1 < 2 * G:
                a_nxt, r_nxt = 1, m + 1 - G
            else:
                a_nxt, r_nxt = 0, 0

            _wait_gather(srcg[a_cur], r_cur, b)

            if m == G - 1:
                _wait_group(2 * d + 1, 1)
            if m == 2 * G - 1:
                @pl.when(d < NG // 2 - 1)
                def _wA():
                    _wait_group(2 * d + 2, 0)

            @pl.when(i + 1 < NCHUNK)
            def _refill():
                @pl.when(i > 0)
                def _drain_prev():
                    _wait_scatter(dstg[a_cur], r_cur, nb)

                _gather(srcg[a_nxt], r_nxt, nb)

            if m == 0:
                _load_group(2 * d + 1, 1, False)
            if m == G:
                @pl.when(d < NG // 2 - 1)
                def _lA():
                    _load_group(2 * d + 2, 0, False)

            _scale(wg[a_cur], r_cur, b)
            _scatter(dstg[a_cur], r_cur, b)
        return 0

    lax.fori_loop(0, NG // 2, _dgroup, 0)

    for b in range(NBUF):
        _wait_scatter(dstA, 0, b)
    plsc.subcore_barrier()

    @pl.when(sid < NS - 1)
    def _wb_main():
        pltpu.sync_copy(acc_sh.at[pl.ds(row_base, ROWS_PT)],
                        out_hbm.at[cid, pl.ds(row_base, ROWS_PT)])

    @pl.when(sid == NS - 1)
    def _wb_last():
        last = (NS - 1) * ROWS_PT
        pltpu.sync_copy(acc_sh.at[pl.ds(last, ROWS_LAST)],
                        out_hbm.at[cid, pl.ds(last, ROWS_LAST)])


def _add_body(a_ref, b_ref, o_ref):
    o_ref[...] = a_ref[...] + b_ref[...]


def _combine(p0, p1):
    grid = 10
    rows = N_NODES // grid
    return pl.pallas_call(
        _add_body,
        grid=(grid,),
        in_specs=[
            pl.BlockSpec((rows, D_OUT), lambda i: (i, 0)),
            pl.BlockSpec((rows, D_OUT), lambda i: (i, 0)),
        ],
        out_specs=pl.BlockSpec((rows, D_OUT), lambda i: (i, 0)),
        out_shape=jax.ShapeDtypeStruct((N_NODES, D_OUT), jnp.float32),
    )(p0, p1)


def kernel(x, edge_index, edge_weight, W):
    edge_index = edge_index.astype(jnp.int32)
    pad = E_PAD - N_EDGES
    src = jnp.concatenate(
        [edge_index[0], jnp.zeros((pad,), jnp.int32)]).reshape(
            NW, NG, G, CHUNK)
    dst = jnp.concatenate(
        [edge_index[1], jnp.zeros((pad,), jnp.int32)]).reshape(
            NW, NG, G, CHUNK)
    ew = jnp.concatenate(
        [edge_weight, jnp.zeros((pad,), jnp.float32)]).reshape(
            NW, NG, G, CHUNK)
    h = _matmul(x, W)
    partials = _sc_edges(src, dst, ew, h)
    return _combine(partials[0], partials[1])
